# baseline (speedup 1.0000x reference)
import numpy as np

import jax
import jax.numpy as jnp
from jax.experimental import pallas as pl
from jax.sharding import Mesh, PartitionSpec as P

_PREC = jax.lax.Precision.DEFAULT
_DN = (((1,), (0,)), ((), ()))
_B = 4
_K = 320


def _mm(a, b):
    return jax.lax.dot_general(a, b, _DN, precision=_PREC,
                               preferred_element_type=jnp.float32)


def _make_body(nb):
    def _body(wav_ref, w0_ref, w1_ref, w2_ref, w3_ref, w4_ref,
              b0_ref, b1_ref, b2_ref, b3_ref, b4_ref, ct_ref, out_ref):
        for b in range(nb):
            x = wav_ref[b]
            h = _mm(x[0:4799], w0_ref[0:5]) + _mm(x[1:4800], w0_ref[5:10])
            h = jnp.maximum(h + b0_ref[...], 0.0)

            f = h[0:4796].reshape(1199, 2048)
            h = _mm(f[0:1198], w1_ref[0:2048]) + _mm(f[1:1199], w1_ref[2048:4096])
            h = jnp.maximum(h + b1_ref[...], 0.0)

            f = h.reshape(599, 1024)
            h = _mm(f[0:598], w2_ref[0:1024]) + _mm(f[1:599], w2_ref[1024:2048])
            h = jnp.maximum(h + b2_ref[...], 0.0)

            f = h.reshape(299, 1024)
            h = _mm(f[0:298], w3_ref[0:1024]) + _mm(f[1:299], w3_ref[1024:2048])
            h = jnp.maximum(h + b3_ref[...], 0.0)

            f = h.reshape(149, 1024)
            h = _mm(f[0:148], w4_ref[0:1024]) + _mm(f[1:149], w4_ref[1024:2048])
            h = jnp.maximum(h + b4_ref[...], 0.0)

            for g in range(2):
                xg = h[:, 256 * g:256 * (g + 1)]
                ct = ct_ref[g]
                x2 = jnp.sum(xg * xg, axis=1, keepdims=True)
                c2 = jnp.sum(ct * ct, axis=0, keepdims=True)
                dist = (x2 - 2.0 * _mm(xg, ct)) + c2
                m = jnp.min(dist, axis=1, keepdims=True)
                k_iota = jax.lax.broadcasted_iota(jnp.int32, dist.shape, 1)
                idx = jnp.min(jnp.where(dist == m, k_iota, jnp.int32(_K)),
                              axis=1)
                out_ref[b, g] = idx
    return _body


def _shard_call(wavf, *rest):
    nb = wavf.shape[0]
    return pl.pallas_call(
        _make_body(nb),
        out_shape=jax.ShapeDtypeStruct((nb, 2, 148), jnp.int32),
    )(wavf, *rest)


def kernel(wav_input, conv_w0, conv_b0, conv_w1, conv_b1, conv_w2, conv_b2,
           conv_w3, conv_b3, conv_w4, conv_b4, codebook):
    wavf = wav_input.reshape(_B, 4800, 5)
    w0 = conv_w0[:, 0, :].T
    w1 = conv_w1.transpose(2, 1, 0).reshape(4096, 512)
    w2 = conv_w2.transpose(2, 1, 0).reshape(2048, 512)
    w3 = conv_w3.transpose(2, 1, 0).reshape(2048, 512)
    w4 = conv_w4.transpose(2, 1, 0).reshape(2048, 512)
    ct = codebook.transpose(0, 2, 1)
    bs = [b.reshape(1, 512) for b in
          (conv_b0, conv_b1, conv_b2, conv_b3, conv_b4)]

    devs = jax.devices()
    nd = 4 if len(devs) >= 4 else (2 if len(devs) >= 2 else 1)
    mesh = Mesh(np.asarray(devs[:nd]), ('b',))
    rep = (P(),) * 11
    f = jax.shard_map(_shard_call, mesh=mesh,
                      in_specs=(P('b'),) + rep,
                      out_specs=P('b'), check_vma=False)
    out = f(wavf, w0, w1, w2, w3, w4, *bs, ct)
    return out.transpose(0, 2, 1).reshape(_B, 296)


# speedup vs baseline: 4.3915x; 4.3915x over previous
import jax
import jax.numpy as jnp
from jax.experimental import pallas as pl

_PREC = jax.lax.Precision.DEFAULT
_DN = (((1,), (0,)), ((), ()))
_B = 4
_K = 320


def _mm(a, b):
    return jax.lax.dot_general(a, b, _DN, precision=_PREC,
                               preferred_element_type=jnp.float32)


def _zpad1(h):
    return jnp.concatenate([h, jnp.zeros((1, h.shape[1]), jnp.float32)], 0)


def _head_body(wav_ref, w0_ref, w1_ref, b0_ref, b1_ref, out_ref):
    x = wav_ref[0]
    h = _mm(x[0:4799], w0_ref[0:5]) + _mm(x[1:4800], w0_ref[5:10])
    h = jnp.maximum(h + b0_ref[...], 0.0)

    f = h[0:4796].reshape(1199, 2048)
    h = _mm(f[0:1198], w1_ref[0:2048]) + _mm(f[1:1199], w1_ref[2048:4096])
    h = jnp.maximum(h + b1_ref[...], 0.0)
    out_ref[0, 0:1198] = h
    out_ref[0, 1198:1200] = jnp.zeros((2, 512), jnp.float32)


def _tail_body(f_ref, w2_ref, w3_ref, w4_ref, b2_ref, b3_ref, b4_ref,
               ct_ref, out_ref):
    f = f_ref[...]
    h = _mm(f[0:2399], w2_ref[0:1024]) + _mm(f[1:2400], w2_ref[1024:2048])
    h = jnp.maximum(h + b2_ref[...], 0.0)

    f = _zpad1(h).reshape(1200, 1024)
    h = _mm(f[0:1199], w3_ref[0:1024]) + _mm(f[1:1200], w3_ref[1024:2048])
    h = jnp.maximum(h + b3_ref[...], 0.0)

    f = _zpad1(h).reshape(600, 1024)
    h = _mm(f[0:599], w4_ref[0:1024]) + _mm(f[1:600], w4_ref[1024:2048])
    h = jnp.maximum(h + b4_ref[...], 0.0)
    h = _zpad1(h)

    for g in range(2):
        xg = h[:, 256 * g:256 * (g + 1)]
        ct = ct_ref[g]
        x2 = jnp.sum(xg * xg, axis=1, keepdims=True)
        c2 = jnp.sum(ct * ct, axis=0, keepdims=True)
        dist = (x2 - 2.0 * _mm(xg, ct)) + c2
        m = jnp.min(dist, axis=1, keepdims=True)
        k_iota = jax.lax.broadcasted_iota(jnp.int32, dist.shape, 1)
        idx = jnp.min(jnp.where(dist == m, k_iota, jnp.int32(_K)), axis=1)
        out_ref[g] = idx


def kernel(wav_input, conv_w0, conv_b0, conv_w1, conv_b1, conv_w2, conv_b2,
           conv_w3, conv_b3, conv_w4, conv_b4, codebook):
    wavf = wav_input.reshape(_B, 4800, 5)
    w0 = conv_w0[:, 0, :].T
    w1 = conv_w1.transpose(2, 1, 0).reshape(4096, 512)
    w2 = conv_w2.transpose(2, 1, 0).reshape(2048, 512)
    w3 = conv_w3.transpose(2, 1, 0).reshape(2048, 512)
    w4 = conv_w4.transpose(2, 1, 0).reshape(2048, 512)
    ct = codebook.transpose(0, 2, 1)
    b0, b1, b2, b3, b4 = [b.reshape(1, 512) for b in
                          (conv_b0, conv_b1, conv_b2, conv_b3, conv_b4)]

    h1 = pl.pallas_call(
        _head_body,
        grid=(_B,),
        in_specs=[
            pl.BlockSpec((1, 4800, 5), lambda b: (b, 0, 0)),
            pl.BlockSpec((10, 512), lambda b: (0, 0)),
            pl.BlockSpec((4096, 512), lambda b: (0, 0)),
            pl.BlockSpec((1, 512), lambda b: (0, 0)),
            pl.BlockSpec((1, 512), lambda b: (0, 0)),
        ],
        out_specs=pl.BlockSpec((1, 1200, 512), lambda b: (b, 0, 0)),
        out_shape=jax.ShapeDtypeStruct((_B, 1200, 512), jnp.float32),
    )(wavf, w0, w1, b0, b1)

    f2 = h1.reshape(2400, 1024)
    out = pl.pallas_call(
        _tail_body,
        out_shape=jax.ShapeDtypeStruct((2, 600), jnp.int32),
    )(f2, w2, w3, w4, b2, b3, b4, ct)
    cols = jnp.stack([out[:, 150 * b:150 * b + 148] for b in range(_B)], 0)
    return cols.transpose(0, 2, 1).reshape(_B, 296)


# speedup vs baseline: 4.8675x; 1.1084x over previous
import jax
import jax.numpy as jnp
from jax.experimental import pallas as pl

_PREC = jax.lax.Precision.DEFAULT
_DN = (((1,), (0,)), ((), ()))
_B = 4
_K = 320


def _mm(a, b):
    return jax.lax.dot_general(a, b, _DN, precision=_PREC,
                               preferred_element_type=jnp.float32)


def _body(wav_ref, w0_ref, w1_ref, w2_ref, w3_ref, w4_ref,
          b0_ref, b1_ref, b2_ref, b3_ref, b4_ref, ct_ref, out_ref):
    for b in range(_B):
        x = wav_ref[b]
        xx = jnp.concatenate([x[0:4799], x[1:4800]], axis=1)
        h = _mm(xx, w0_ref[...])
        h = jnp.maximum(h + b0_ref[...], 0.0)

        f = h[0:4796].reshape(1199, 2048)
        h = _mm(f[0:1198], w1_ref[0:2048]) + _mm(f[1:1199], w1_ref[2048:4096])
        h = jnp.maximum(h + b1_ref[...], 0.0)

        f = h.reshape(599, 1024)
        h = _mm(f[0:598], w2_ref[0:1024]) + _mm(f[1:599], w2_ref[1024:2048])
        h = jnp.maximum(h + b2_ref[...], 0.0)

        f = h.reshape(299, 1024)
        h = _mm(f[0:298], w3_ref[0:1024]) + _mm(f[1:299], w3_ref[1024:2048])
        h = jnp.maximum(h + b3_ref[...], 0.0)

        f = h.reshape(149, 1024)
        h = _mm(f[0:148], w4_ref[0:1024]) + _mm(f[1:149], w4_ref[1024:2048])
        h = jnp.maximum(h + b4_ref[...], 0.0)

        for g in range(2):
            xg = h[:, 256 * g:256 * (g + 1)]
            ct = ct_ref[g]
            x2 = jnp.sum(xg * xg, axis=1, keepdims=True)
            c2 = jnp.sum(ct * ct, axis=0, keepdims=True)
            dist = (x2 - 2.0 * _mm(xg, ct)) + c2
            m = jnp.min(dist, axis=1, keepdims=True)
            k_iota = jax.lax.broadcasted_iota(jnp.int32, dist.shape, 1)
            idx = jnp.min(jnp.where(dist == m, k_iota, jnp.int32(_K)), axis=1)
            out_ref[b, g] = idx


def kernel(wav_input, conv_w0, conv_b0, conv_w1, conv_b1, conv_w2, conv_b2,
           conv_w3, conv_b3, conv_w4, conv_b4, codebook):
    wavf = wav_input.reshape(_B, 4800, 5)
    w0 = conv_w0[:, 0, :].T
    w1 = conv_w1.transpose(2, 1, 0).reshape(4096, 512)
    w2 = conv_w2.transpose(2, 1, 0).reshape(2048, 512)
    w3 = conv_w3.transpose(2, 1, 0).reshape(2048, 512)
    w4 = conv_w4.transpose(2, 1, 0).reshape(2048, 512)
    ct = codebook.transpose(0, 2, 1)
    bs = [b.reshape(1, 512) for b in
          (conv_b0, conv_b1, conv_b2, conv_b3, conv_b4)]
    out = pl.pallas_call(
        _body,
        out_shape=jax.ShapeDtypeStruct((_B, 2, 148), jnp.int32),
    )(wavf, w0, w1, w2, w3, w4, *bs, ct)
    return out.transpose(0, 2, 1).reshape(_B, 296)
